# baseline (speedup 1.0000x reference)
import functools

import jax
import jax.numpy as jnp
from jax import lax
from jax.experimental import pallas as pl
from jax.experimental.pallas import tpu as pltpu
from jax.experimental.pallas import tpu_sc as plsc

N = 10000
E = 320000
HID = 128
HEADS = 8
DH = HID // HEADS
CLAMP = 5.0

NC = 2
NS = 16
NW = NC * NS
EPW = E // NW
CH = 128
NP = 10240
RPS = NP // NS
ZR = 64
NFULL = EPW // CH
TAIL = EPW - NFULL * CH
PAIRS = NFULL // 2 - 1
SCH = 64
SNFULL = EPW // SCH
STAIL = EPW - SNFULL * SCH
SPAIRS = SNFULL // 2 - 1

_f32 = jnp.float32


def _ln(x, w, b, eps=1e-5):
    mu = jnp.mean(x, axis=-1, keepdims=True)
    var = jnp.mean((x - mu) ** 2, axis=-1, keepdims=True)
    return (x - mu) / jnp.sqrt(var + eps) * w + b


def _qkv_body(x_ref, w_ref, b_ref, q_ref, kv_ref):
    qkv = jnp.dot(x_ref[...], w_ref[...], preferred_element_type=_f32) + b_ref[...]
    q_ref[...] = qkv[:, :HID]
    kv_ref[...] = qkv[:, HID:].astype(jnp.bfloat16)


def _qkv_call(x, wT, b2):
    return pl.pallas_call(
        _qkv_body,
        out_shape=(
            jax.ShapeDtypeStruct((N, HID), _f32),
            jax.ShapeDtypeStruct((N, 2 * HID), jnp.bfloat16),
        ),
    )(x, wT, b2)


def _gather_body(q_hbm, kv_hbm, dst_hbm, src_hbm, qd_hbm, kvs_hbm,
                 idx_d0, idx_s0, idx_d1, idx_s1, qb0, kvb0, qb1, kvb1,
                 gsem0, gsem1, wsem0, wsem1):
    wid = lax.axis_index("s") * NC + lax.axis_index("c")
    base = wid * EPW
    sets = ((idx_d0, idx_s0, qb0, kvb0, gsem0, wsem0),
            (idx_d1, idx_s1, qb1, kvb1, gsem1, wsem1))

    def issue_gather(c, k):
        idx_d, idx_s, qb, kvb, gsem, _ = sets[k]
        off = base + c * CH
        pltpu.sync_copy(dst_hbm.at[pl.ds(off, CH)], idx_d)
        pltpu.sync_copy(src_hbm.at[pl.ds(off, CH)], idx_s)
        pltpu.async_copy(q_hbm.at[idx_d], qb, gsem)
        pltpu.async_copy(kv_hbm.at[idx_s], kvb, gsem)

    def wait_gather(k):
        idx_d, idx_s, qb, kvb, gsem, _ = sets[k]
        pltpu.make_async_copy(q_hbm.at[idx_d], qb, gsem).wait()
        pltpu.make_async_copy(kv_hbm.at[idx_s], kvb, gsem).wait()

    def issue_wb(c, k):
        _, _, qb, kvb, _, wsem = sets[k]
        off = base + c * CH
        pltpu.async_copy(qb, qd_hbm.at[pl.ds(off, CH)], wsem)
        pltpu.async_copy(kvb, kvs_hbm.at[pl.ds(off, CH)], wsem)

    def wait_wb(c, k):
        _, _, qb, kvb, _, wsem = sets[k]
        off = base + c * CH
        pltpu.make_async_copy(qb, qd_hbm.at[pl.ds(off, CH)], wsem).wait()
        pltpu.make_async_copy(kvb, kvs_hbm.at[pl.ds(off, CH)], wsem).wait()

    issue_gather(0, 0)

    def pair(j, carry):
        a = 2 * j
        b = a + 1
        issue_gather(b, 1)
        wait_gather(0)
        issue_wb(a, 0)
        wait_gather(1)
        issue_wb(b, 1)
        wait_wb(a, 0)
        issue_gather(a + 2, 0)
        wait_wb(b, 1)
        return carry

    lax.fori_loop(0, PAIRS, pair, 0)
    issue_gather(NFULL - 1, 1)
    wait_gather(0)
    issue_wb(NFULL - 2, 0)
    wait_gather(1)
    issue_wb(NFULL - 1, 1)
    wait_wb(NFULL - 2, 0)
    toff = base + NFULL * CH
    pltpu.sync_copy(dst_hbm.at[pl.ds(toff, TAIL)], idx_d0.at[pl.ds(0, TAIL)])
    pltpu.sync_copy(src_hbm.at[pl.ds(toff, TAIL)], idx_s0.at[pl.ds(0, TAIL)])
    pltpu.async_copy(q_hbm.at[idx_d0.at[pl.ds(0, TAIL)]],
                     qb0.at[pl.ds(0, TAIL)], gsem0).wait()
    pltpu.async_copy(kv_hbm.at[idx_s0.at[pl.ds(0, TAIL)]],
                     kvb0.at[pl.ds(0, TAIL)], gsem0).wait()
    pltpu.sync_copy(qb0.at[pl.ds(0, TAIL)], qd_hbm.at[pl.ds(toff, TAIL)])
    pltpu.sync_copy(kvb0.at[pl.ds(0, TAIL)], kvs_hbm.at[pl.ds(toff, TAIL)])
    wait_wb(NFULL - 1, 1)


def _gather_call(q, kv, dst, src):
    mesh = plsc.VectorSubcoreMesh(core_axis_name="c", subcore_axis_name="s")
    fn = pl.kernel(
        _gather_body,
        out_type=(
            jax.ShapeDtypeStruct((E, HID), _f32),
            jax.ShapeDtypeStruct((E, HID), jnp.int32),
        ),
        mesh=mesh,
        scratch_types=[
            pltpu.VMEM((CH,), jnp.int32),
            pltpu.VMEM((CH,), jnp.int32),
            pltpu.VMEM((CH,), jnp.int32),
            pltpu.VMEM((CH,), jnp.int32),
            pltpu.VMEM((CH, HID), _f32),
            pltpu.VMEM((CH, HID), jnp.int32),
            pltpu.VMEM((CH, HID), _f32),
            pltpu.VMEM((CH, HID), jnp.int32),
            pltpu.SemaphoreType.DMA,
            pltpu.SemaphoreType.DMA,
            pltpu.SemaphoreType.DMA,
            pltpu.SemaphoreType.DMA,
        ],
    )
    return fn(q, kv, dst, src)


def _edge_body(e_ref, qd_ref, kvs_ref, ewT_ref, ewb_ref, A_ref, R_ref,
               c1T_ref, c1b_ref, c2T_ref, c2b_ref,
               ln1cw_ref, ln1cb_ref, ln2cw_ref, ln2cb_ref,
               conn_ref, pack_ref, wgt_ref):
    e_blk = e_ref[...]
    eh = jnp.dot(e_blk, ewT_ref[...], preferred_element_type=_f32) + ewb_ref[...]
    ew = eh[:, :HID]
    eb = eh[:, HID:]
    qd = qd_ref[...].astype(_f32)
    kvs = kvs_ref[...].astype(_f32)
    conn1 = (qd + kvs[:, :HID]) * ew
    conn2 = jnp.sign(conn1) * jnp.sqrt(jnp.abs(conn1))
    conn = jnp.maximum(conn2 + eb, 0.0)
    score = jnp.dot(conn, A_ref[...], preferred_element_type=_f32)
    wgt = jnp.exp(jnp.clip(score, -CLAMP, CLAMP))
    connc1 = jnp.dot(conn, c1T_ref[...], preferred_element_type=_f32) + c1b_ref[...]
    msg = kvs[:, HID:] + connc1
    contrib = msg * jnp.dot(wgt, R_ref[...], preferred_element_type=_f32)
    pack_ref[...] = contrib
    wgt_ref[...] = jnp.concatenate([wgt, jnp.zeros_like(wgt)], axis=1)
    cl = jnp.maximum(_ln(connc1, ln1cw_ref[...], ln1cb_ref[...]), 0.0)
    co = jnp.dot(cl, c2T_ref[...], preferred_element_type=_f32) + c2b_ref[...] + e_blk
    conn_ref[...] = _ln(co, ln2cw_ref[...], ln2cb_ref[...])


def _edge_call(e, qd, kvs, ewT, ewb, A, R, c1T, c1b, c2T, c2b,
               ln1cw, ln1cb, ln2cw, ln2cb):
    B = 2000
    grid = E // B
    full = lambda shape: pl.BlockSpec(shape, lambda i: (0, 0))
    return pl.pallas_call(
        _edge_body,
        grid=(grid,),
        in_specs=[
            pl.BlockSpec((B, HID), lambda i: (i, 0)),
            pl.BlockSpec((B, HID), lambda i: (i, 0)),
            pl.BlockSpec((B, 2 * HID), lambda i: (i, 0)),
            full((HID, 2 * HID)),
            full((1, 2 * HID)),
            full((HID, HEADS)),
            full((HEADS, HID)),
            full((HID, HID)),
            full((1, HID)),
            full((HID, HID)),
            full((1, HID)),
            full((1, HID)),
            full((1, HID)),
            full((1, HID)),
            full((1, HID)),
        ],
        out_specs=[
            pl.BlockSpec((B, HID), lambda i: (i, 0)),
            pl.BlockSpec((B, HID), lambda i: (i, 0)),
            pl.BlockSpec((B, 2 * HEADS), lambda i: (i, 0)),
        ],
        out_shape=(
            jax.ShapeDtypeStruct((E, HID), _f32),
            jax.ShapeDtypeStruct((E, HID), _f32),
            jax.ShapeDtypeStruct((E, 2 * HEADS), _f32),
        ),
    )(e, qd, kvs, ewT, ewb, A, R, c1T, c1b, c2T, c2b,
      ln1cw, ln1cb, ln2cw, ln2cb)


def _scatter_body(pack_hbm, wgt_hbm, dst_hbm, agg_hbm,
                  idx0, idx1, idx_t, vbuf0, vbuf1, wbuf0, wbuf1,
                  zbuf, shared, lsem0, lsem1):
    c = lax.axis_index("c")
    s = lax.axis_index("s")
    wid = s * NC + c
    base = wid * EPW
    isets = (idx0, idx1)
    lsems = (lsem0, lsem1)
    vsets = (vbuf0, vbuf1)
    wsets = (wbuf0, wbuf1)

    def zrow(i, carry):
        for j in range(HID // 16):
            zbuf[i, pl.ds(j * 16, 16)] = jnp.zeros((16,), _f32)
        return carry

    lax.fori_loop(0, ZR, zrow, 0)

    def zero_shared():
        def zcp(k, carry):
            pltpu.sync_copy(zbuf, shared.at[pl.ds(s * RPS + k * ZR, ZR)])
            return carry
        lax.fori_loop(0, RPS // ZR, zcp, 0)

    def phase(val_hbm, vbufs, scatter_fn, out_row):
        def issue(ch, k):
            off = base + ch * SCH
            pltpu.async_copy(dst_hbm.at[pl.ds(off, SCH)], isets[k], lsems[k])
            pltpu.async_copy(val_hbm.at[pl.ds(off, SCH)], vbufs[k], lsems[k])

        def wait(ch, k):
            off = base + ch * SCH
            pltpu.make_async_copy(
                dst_hbm.at[pl.ds(off, SCH)], isets[k], lsems[k]).wait()
            pltpu.make_async_copy(
                val_hbm.at[pl.ds(off, SCH)], vbufs[k], lsems[k]).wait()

        zero_shared()
        plsc.subcore_barrier()
        issue(0, 0)

        def pair(j, carry):
            a = 2 * j
            issue(a + 1, 1)
            wait(a, 0)
            scatter_fn(0)
            issue(a + 2, 0)
            wait(a + 1, 1)
            scatter_fn(1)
            return carry

        lax.fori_loop(0, SPAIRS, pair, 0)
        issue(SNFULL - 1, 1)
        wait(SNFULL - 2, 0)
        scatter_fn(0)
        wait(SNFULL - 1, 1)
        scatter_fn(1)
        toff = base + SNFULL * SCH
        pltpu.sync_copy(dst_hbm.at[pl.ds(toff, STAIL)], idx_t)
        pltpu.sync_copy(val_hbm.at[pl.ds(toff, STAIL)],
                        vbufs[0].at[pl.ds(0, STAIL)])
        scatter_fn(0, STAIL)
        plsc.subcore_barrier()
        pltpu.sync_copy(shared.at[pl.ds(s * RPS, RPS)],
                        agg_hbm.at[pl.ds(out_row + s * RPS, RPS)])
        plsc.subcore_barrier()

    def scat1(k, tail=0):
        if tail:
            pltpu.sync_copy(vsets[0].at[pl.ds(0, tail)],
                            shared.at[idx_t], add=True)
        else:
            pltpu.sync_copy(vsets[k], shared.at[isets[k]], add=True)

    phase(pack_hbm, vsets, scat1, c * 2 * NP)

    def zpad(i, carry):
        for j in range(HID // 16):
            vbuf0[i, pl.ds(j * 16, 16)] = jnp.zeros((16,), _f32)
        return carry

    lax.fori_loop(0, SCH, zpad, 0)

    def scat2(k, tail=0):
        n = tail if tail else SCH
        for r in range(n):
            vbuf0[r, pl.ds(0, 16)] = wsets[k][r, pl.ds(0, 16)]
        if tail:
            pltpu.sync_copy(vbuf0.at[pl.ds(0, tail)], shared.at[idx_t], add=True)
        else:
            pltpu.sync_copy(vbuf0, shared.at[isets[k]], add=True)

    phase(wgt_hbm, wsets, scat2, (c * 2 + 1) * NP)


def _scatter_call(pack, wgt, dst):
    mesh = plsc.VectorSubcoreMesh(core_axis_name="c", subcore_axis_name="s")
    fn = pl.kernel(
        _scatter_body,
        out_type=jax.ShapeDtypeStruct((NC * 2 * NP, HID), _f32),
        mesh=mesh,
        scratch_types=[
            pltpu.VMEM((SCH,), jnp.int32),
            pltpu.VMEM((SCH,), jnp.int32),
            pltpu.VMEM((STAIL,), jnp.int32),
            pltpu.VMEM((SCH, HID), _f32),
            pltpu.VMEM((SCH, HID), _f32),
            pltpu.VMEM((SCH, 2 * HEADS), _f32),
            pltpu.VMEM((SCH, 2 * HEADS), _f32),
            pltpu.VMEM((ZR, HID), _f32),
            pltpu.VMEM_SHARED((NP, HID), _f32),
            pltpu.SemaphoreType.DMA,
            pltpu.SemaphoreType.DMA,
        ],
    )
    return fn(pack, wgt, dst)


def _node_body(x_ref, agg_ref, R_ref, f1T_ref, f1b_ref, f2T_ref, f2b_ref,
               ln1w_ref, ln1b_ref, ln2w_ref, ln2b_ref, h_ref):
    agg = agg_ref[0, 0] + agg_ref[1, 0]
    den8 = agg_ref[0, 1, :, :HEADS] + agg_ref[1, 1, :, :HEADS]
    den = jnp.dot(den8, R_ref[...], preferred_element_type=_f32)
    h = x_ref[...] + agg / (den + 1e-16)
    hres = h
    h = _ln(h, ln1w_ref[...], ln1b_ref[...])
    h = jnp.maximum(jnp.dot(h, f1T_ref[...], preferred_element_type=_f32)
                    + f1b_ref[...], 0.0)
    h = jnp.dot(h, f2T_ref[...], preferred_element_type=_f32) + f2b_ref[...] + hres
    h_ref[...] = _ln(h, ln2w_ref[...], ln2b_ref[...])


def _node_call(x, agg2, R, f1T, f1b, f2T, f2b, ln1w, ln1b, ln2w, ln2b):
    Bn = 1000
    grid = N // Bn
    full = lambda shape: pl.BlockSpec(shape, lambda i: (0,) * len(shape))
    return pl.pallas_call(
        _node_body,
        grid=(grid,),
        in_specs=[
            pl.BlockSpec((Bn, HID), lambda i: (i, 0)),
            pl.BlockSpec((2, 2, Bn, HID), lambda i: (0, 0, i, 0)),
            full((HEADS, HID)),
            full((HID, 2 * HID)),
            full((1, 2 * HID)),
            full((2 * HID, HID)),
            full((1, HID)),
            full((1, HID)),
            full((1, HID)),
            full((1, HID)),
            full((1, HID)),
        ],
        out_specs=pl.BlockSpec((Bn, HID), lambda i: (i, 0)),
        out_shape=jax.ShapeDtypeStruct((N, HID), _f32),
    )(x, agg2, R, f1T, f1b, f2T, f2b, ln1w, ln1b, ln2w, ln2b)


def kernel(x, e, edge_index, qkv_w, qkv_b, ew_w, ew_b, aw, c1_w, c1_b,
           c2_w, c2_b, f1_w, f1_b, f2_w, f2_b, ln1h_w, ln1h_b, ln2h_w,
           ln2h_b, ln1c_w, ln1c_b, ln2c_w, ln2c_b):
    dst = edge_index[0]
    src = edge_index[1]
    r2 = lambda v: v.reshape(1, -1)
    awT = aw[:, :, 0].T
    A = (awT[:, :, None] * jnp.eye(HEADS, dtype=_f32)[:, None, :]).reshape(HID, HEADS)
    R = jnp.repeat(jnp.eye(HEADS, dtype=_f32), DH, axis=1)

    q, kv = _qkv_call(x, qkv_w.T, r2(qkv_b))
    kvi = lax.bitcast_convert_type(kv.reshape(N, HID, 2), jnp.int32)
    qd, kvsi = _gather_call(q, kvi, dst, src)
    kvs = lax.bitcast_convert_type(kvsi, jnp.bfloat16).reshape(E, 2 * HID)
    conn, pack, wgt = _edge_call(
        e, qd, kvs, ew_w.T, r2(ew_b), A, R, c1_w.T, r2(c1_b), c2_w.T,
        r2(c2_b), r2(ln1c_w), r2(ln1c_b), r2(ln2c_w), r2(ln2c_b))
    agg2 = _scatter_call(pack, wgt, dst).reshape(NC, 2, NP, HID)
    h = _node_call(x, agg2, R, f1_w.T, r2(f1_b), f2_w.T, r2(f2_b),
                   r2(ln1h_w), r2(ln1h_b), r2(ln2h_w), r2(ln2h_b))
    return h, conn


# speedup vs baseline: 2.4288x; 2.4288x over previous
import functools

import jax
import jax.numpy as jnp
from jax import lax
from jax.experimental import pallas as pl
from jax.experimental.pallas import tpu as pltpu
from jax.experimental.pallas import tpu_sc as plsc

N = 10000
E = 320000
HID = 128
HEADS = 8
DH = HID // HEADS
CLAMP = 5.0

NC = 2
NS = 16
NW = NC * NS
EPW = E // NW
CH = 128
NP = 10240
RPS = NP // NS
ZR = 64
NFULL = EPW // CH
TAIL = EPW - NFULL * CH
PAIRS = NFULL // 2 - 1
SCH = 64
SNFULL = EPW // SCH
STAIL = EPW - SNFULL * SCH
SPAIRS = SNFULL // 2 - 1

_f32 = jnp.float32


def _ln(x, w, b, eps=1e-5):
    mu = jnp.mean(x, axis=-1, keepdims=True)
    var = jnp.mean((x - mu) ** 2, axis=-1, keepdims=True)
    return (x - mu) / jnp.sqrt(var + eps) * w + b


def _qkv_body(x_ref, w_ref, b_ref, q_ref, kv_ref):
    qkv = jnp.dot(x_ref[...], w_ref[...], preferred_element_type=_f32) + b_ref[...]
    q_ref[...] = qkv[:, :HID]
    ku = lax.bitcast_convert_type(
        qkv[:, HID:2 * HID].astype(jnp.bfloat16), jnp.uint16).astype(jnp.uint32)
    vu = lax.bitcast_convert_type(
        qkv[:, 2 * HID:].astype(jnp.bfloat16), jnp.uint16).astype(jnp.uint32)
    kv_ref[...] = lax.bitcast_convert_type((ku << 16) | vu, jnp.int32)


def _qkv_call(x, wT, b2):
    return pl.pallas_call(
        _qkv_body,
        out_shape=(
            jax.ShapeDtypeStruct((N, HID), _f32),
            jax.ShapeDtypeStruct((N, HID), jnp.int32),
        ),
    )(x, wT, b2)


def _gather_body(q_hbm, kv_hbm, dst_hbm, src_hbm, qd_hbm, kvs_hbm,
                 idx_d0, idx_s0, idx_d1, idx_s1, qb0, kvb0, qb1, kvb1,
                 gsem0, gsem1, wsem0, wsem1):
    wid = lax.axis_index("s") * NC + lax.axis_index("c")
    base = wid * EPW
    sets = ((idx_d0, idx_s0, qb0, kvb0, gsem0, wsem0),
            (idx_d1, idx_s1, qb1, kvb1, gsem1, wsem1))

    def issue_gather(c, k):
        idx_d, idx_s, qb, kvb, gsem, _ = sets[k]
        off = base + c * CH
        pltpu.sync_copy(dst_hbm.at[pl.ds(off, CH)], idx_d)
        pltpu.sync_copy(src_hbm.at[pl.ds(off, CH)], idx_s)
        pltpu.async_copy(q_hbm.at[idx_d], qb, gsem)
        pltpu.async_copy(kv_hbm.at[idx_s], kvb, gsem)

    def wait_gather(k):
        idx_d, idx_s, qb, kvb, gsem, _ = sets[k]
        pltpu.make_async_copy(q_hbm.at[idx_d], qb, gsem).wait()
        pltpu.make_async_copy(kv_hbm.at[idx_s], kvb, gsem).wait()

    def issue_wb(c, k):
        _, _, qb, kvb, _, wsem = sets[k]
        off = base + c * CH
        pltpu.async_copy(qb, qd_hbm.at[pl.ds(off, CH)], wsem)
        pltpu.async_copy(kvb, kvs_hbm.at[pl.ds(off, CH)], wsem)

    def wait_wb(c, k):
        _, _, qb, kvb, _, wsem = sets[k]
        off = base + c * CH
        pltpu.make_async_copy(qb, qd_hbm.at[pl.ds(off, CH)], wsem).wait()
        pltpu.make_async_copy(kvb, kvs_hbm.at[pl.ds(off, CH)], wsem).wait()

    issue_gather(0, 0)

    def pair(j, carry):
        a = 2 * j
        b = a + 1
        issue_gather(b, 1)
        wait_gather(0)
        issue_wb(a, 0)
        wait_gather(1)
        issue_wb(b, 1)
        wait_wb(a, 0)
        issue_gather(a + 2, 0)
        wait_wb(b, 1)
        return carry

    lax.fori_loop(0, PAIRS, pair, 0)
    issue_gather(NFULL - 1, 1)
    wait_gather(0)
    issue_wb(NFULL - 2, 0)
    wait_gather(1)
    issue_wb(NFULL - 1, 1)
    wait_wb(NFULL - 2, 0)
    toff = base + NFULL * CH
    pltpu.sync_copy(dst_hbm.at[pl.ds(toff, TAIL)], idx_d0.at[pl.ds(0, TAIL)])
    pltpu.sync_copy(src_hbm.at[pl.ds(toff, TAIL)], idx_s0.at[pl.ds(0, TAIL)])
    pltpu.async_copy(q_hbm.at[idx_d0.at[pl.ds(0, TAIL)]],
                     qb0.at[pl.ds(0, TAIL)], gsem0).wait()
    pltpu.async_copy(kv_hbm.at[idx_s0.at[pl.ds(0, TAIL)]],
                     kvb0.at[pl.ds(0, TAIL)], gsem0).wait()
    pltpu.sync_copy(qb0.at[pl.ds(0, TAIL)], qd_hbm.at[pl.ds(toff, TAIL)])
    pltpu.sync_copy(kvb0.at[pl.ds(0, TAIL)], kvs_hbm.at[pl.ds(toff, TAIL)])
    wait_wb(NFULL - 1, 1)


def _gather_call(q, kv, dst, src):
    mesh = plsc.VectorSubcoreMesh(core_axis_name="c", subcore_axis_name="s")
    fn = pl.kernel(
        _gather_body,
        out_type=(
            jax.ShapeDtypeStruct((E, HID), _f32),
            jax.ShapeDtypeStruct((E, HID), jnp.int32),
        ),
        mesh=mesh,
        scratch_types=[
            pltpu.VMEM((CH,), jnp.int32),
            pltpu.VMEM((CH,), jnp.int32),
            pltpu.VMEM((CH,), jnp.int32),
            pltpu.VMEM((CH,), jnp.int32),
            pltpu.VMEM((CH, HID), _f32),
            pltpu.VMEM((CH, HID), jnp.int32),
            pltpu.VMEM((CH, HID), _f32),
            pltpu.VMEM((CH, HID), jnp.int32),
            pltpu.SemaphoreType.DMA,
            pltpu.SemaphoreType.DMA,
            pltpu.SemaphoreType.DMA,
            pltpu.SemaphoreType.DMA,
        ],
    )
    return fn(q, kv, dst, src)


def _edge_body(e_ref, qd_ref, kvs_ref, ewT_ref, ewb_ref, A_ref, R_ref,
               c1T_ref, c1b_ref, c2T_ref, c2b_ref,
               ln1cw_ref, ln1cb_ref, ln2cw_ref, ln2cb_ref,
               conn_ref, pack_ref, wgt_ref):
    e_blk = e_ref[...]
    eh = jnp.dot(e_blk, ewT_ref[...], preferred_element_type=_f32) + ewb_ref[...]
    ew = eh[:, :HID]
    eb = eh[:, HID:]
    qd = qd_ref[...]
    kvu = lax.bitcast_convert_type(kvs_ref[...], jnp.uint32)
    ks = lax.bitcast_convert_type(
        (kvu >> 16).astype(jnp.uint16), jnp.bfloat16).astype(_f32)
    vs = lax.bitcast_convert_type(
        (kvu & 0xFFFF).astype(jnp.uint16), jnp.bfloat16).astype(_f32)
    conn1 = (qd + ks) * ew
    conn2 = jnp.sign(conn1) * jnp.sqrt(jnp.abs(conn1))
    conn = jnp.maximum(conn2 + eb, 0.0)
    score = jnp.dot(conn, A_ref[...], preferred_element_type=_f32)
    wgt = jnp.exp(jnp.clip(score, -CLAMP, CLAMP))
    connc1 = jnp.dot(conn, c1T_ref[...], preferred_element_type=_f32) + c1b_ref[...]
    msg = vs + connc1
    contrib = msg * jnp.dot(wgt, R_ref[...], preferred_element_type=_f32)
    pack_ref[...] = contrib
    wgt_ref[...] = jnp.concatenate([wgt, jnp.zeros_like(wgt)], axis=1)
    cl = jnp.maximum(_ln(connc1, ln1cw_ref[...], ln1cb_ref[...]), 0.0)
    co = jnp.dot(cl, c2T_ref[...], preferred_element_type=_f32) + c2b_ref[...] + e_blk
    conn_ref[...] = _ln(co, ln2cw_ref[...], ln2cb_ref[...])


def _edge_call(e, qd, kvs, ewT, ewb, A, R, c1T, c1b, c2T, c2b,
               ln1cw, ln1cb, ln2cw, ln2cb):
    B = 2000
    grid = E // B
    full = lambda shape: pl.BlockSpec(shape, lambda i: (0, 0))
    return pl.pallas_call(
        _edge_body,
        grid=(grid,),
        in_specs=[
            pl.BlockSpec((B, HID), lambda i: (i, 0)),
            pl.BlockSpec((B, HID), lambda i: (i, 0)),
            pl.BlockSpec((B, HID), lambda i: (i, 0)),
            full((HID, 2 * HID)),
            full((1, 2 * HID)),
            full((HID, HEADS)),
            full((HEADS, HID)),
            full((HID, HID)),
            full((1, HID)),
            full((HID, HID)),
            full((1, HID)),
            full((1, HID)),
            full((1, HID)),
            full((1, HID)),
            full((1, HID)),
        ],
        out_specs=[
            pl.BlockSpec((B, HID), lambda i: (i, 0)),
            pl.BlockSpec((B, HID), lambda i: (i, 0)),
            pl.BlockSpec((B, 2 * HEADS), lambda i: (i, 0)),
        ],
        out_shape=(
            jax.ShapeDtypeStruct((E, HID), _f32),
            jax.ShapeDtypeStruct((E, HID), _f32),
            jax.ShapeDtypeStruct((E, 2 * HEADS), _f32),
        ),
    )(e, qd, kvs, ewT, ewb, A, R, c1T, c1b, c2T, c2b,
      ln1cw, ln1cb, ln2cw, ln2cb)


def _scatter_body(pack_hbm, wgt_hbm, dst_hbm, agg_hbm,
                  idx0, idx1, idx_t, vbuf0, vbuf1, wbuf0, wbuf1,
                  zbuf, shared, lsem0, lsem1):
    c = lax.axis_index("c")
    s = lax.axis_index("s")
    wid = s * NC + c
    base = wid * EPW
    isets = (idx0, idx1)
    lsems = (lsem0, lsem1)
    vsets = (vbuf0, vbuf1)
    wsets = (wbuf0, wbuf1)

    def zrow(i, carry):
        for j in range(HID // 16):
            zbuf[i, pl.ds(j * 16, 16)] = jnp.zeros((16,), _f32)
        return carry

    lax.fori_loop(0, ZR, zrow, 0)

    def zero_shared():
        def zcp(k, carry):
            pltpu.sync_copy(zbuf, shared.at[pl.ds(s * RPS + k * ZR, ZR)])
            return carry
        lax.fori_loop(0, RPS // ZR, zcp, 0)

    def phase(val_hbm, vbufs, scatter_fn, out_row):
        def issue(ch, k):
            off = base + ch * SCH
            pltpu.async_copy(dst_hbm.at[pl.ds(off, SCH)], isets[k], lsems[k])
            pltpu.async_copy(val_hbm.at[pl.ds(off, SCH)], vbufs[k], lsems[k])

        def wait(ch, k):
            off = base + ch * SCH
            pltpu.make_async_copy(
                dst_hbm.at[pl.ds(off, SCH)], isets[k], lsems[k]).wait()
            pltpu.make_async_copy(
                val_hbm.at[pl.ds(off, SCH)], vbufs[k], lsems[k]).wait()

        zero_shared()
        plsc.subcore_barrier()
        issue(0, 0)

        def pair(j, carry):
            a = 2 * j
            issue(a + 1, 1)
            wait(a, 0)
            scatter_fn(0)
            issue(a + 2, 0)
            wait(a + 1, 1)
            scatter_fn(1)
            return carry

        lax.fori_loop(0, SPAIRS, pair, 0)
        issue(SNFULL - 1, 1)
        wait(SNFULL - 2, 0)
        scatter_fn(0)
        wait(SNFULL - 1, 1)
        scatter_fn(1)
        toff = base + SNFULL * SCH
        pltpu.sync_copy(dst_hbm.at[pl.ds(toff, STAIL)], idx_t)
        pltpu.sync_copy(val_hbm.at[pl.ds(toff, STAIL)],
                        vbufs[0].at[pl.ds(0, STAIL)])
        scatter_fn(0, STAIL)
        plsc.subcore_barrier()
        pltpu.sync_copy(shared.at[pl.ds(s * RPS, RPS)],
                        agg_hbm.at[pl.ds(out_row + s * RPS, RPS)])
        plsc.subcore_barrier()

    def scat1(k, tail=0):
        if tail:
            pltpu.sync_copy(vsets[0].at[pl.ds(0, tail)],
                            shared.at[idx_t], add=True)
        else:
            pltpu.sync_copy(vsets[k], shared.at[isets[k]], add=True)

    phase(pack_hbm, vsets, scat1, c * 2 * NP)

    def zpad(i, carry):
        for j in range(HID // 16):
            vbuf0[i, pl.ds(j * 16, 16)] = jnp.zeros((16,), _f32)
        return carry

    lax.fori_loop(0, SCH, zpad, 0)

    def scat2(k, tail=0):
        n = tail if tail else SCH
        for r in range(n):
            vbuf0[r, pl.ds(0, 16)] = wsets[k][r, pl.ds(0, 16)]
        if tail:
            pltpu.sync_copy(vbuf0.at[pl.ds(0, tail)], shared.at[idx_t], add=True)
        else:
            pltpu.sync_copy(vbuf0, shared.at[isets[k]], add=True)

    phase(wgt_hbm, wsets, scat2, (c * 2 + 1) * NP)


def _scatter_call(pack, wgt, dst):
    mesh = plsc.VectorSubcoreMesh(core_axis_name="c", subcore_axis_name="s")
    fn = pl.kernel(
        _scatter_body,
        out_type=jax.ShapeDtypeStruct((NC * 2 * NP, HID), _f32),
        mesh=mesh,
        scratch_types=[
            pltpu.VMEM((SCH,), jnp.int32),
            pltpu.VMEM((SCH,), jnp.int32),
            pltpu.VMEM((STAIL,), jnp.int32),
            pltpu.VMEM((SCH, HID), _f32),
            pltpu.VMEM((SCH, HID), _f32),
            pltpu.VMEM((SCH, 2 * HEADS), _f32),
            pltpu.VMEM((SCH, 2 * HEADS), _f32),
            pltpu.VMEM((ZR, HID), _f32),
            pltpu.VMEM_SHARED((NP, HID), _f32),
            pltpu.SemaphoreType.DMA,
            pltpu.SemaphoreType.DMA,
        ],
    )
    return fn(pack, wgt, dst)


def _node_body(x_ref, agg_ref, R_ref, f1T_ref, f1b_ref, f2T_ref, f2b_ref,
               ln1w_ref, ln1b_ref, ln2w_ref, ln2b_ref, h_ref):
    agg = agg_ref[0, 0] + agg_ref[1, 0]
    den8 = agg_ref[0, 1, :, :HEADS] + agg_ref[1, 1, :, :HEADS]
    den = jnp.dot(den8, R_ref[...], preferred_element_type=_f32)
    h = x_ref[...] + agg / (den + 1e-16)
    hres = h
    h = _ln(h, ln1w_ref[...], ln1b_ref[...])
    h = jnp.maximum(jnp.dot(h, f1T_ref[...], preferred_element_type=_f32)
                    + f1b_ref[...], 0.0)
    h = jnp.dot(h, f2T_ref[...], preferred_element_type=_f32) + f2b_ref[...] + hres
    h_ref[...] = _ln(h, ln2w_ref[...], ln2b_ref[...])


def _node_call(x, agg2, R, f1T, f1b, f2T, f2b, ln1w, ln1b, ln2w, ln2b):
    Bn = 1000
    grid = N // Bn
    full = lambda shape: pl.BlockSpec(shape, lambda i: (0,) * len(shape))
    return pl.pallas_call(
        _node_body,
        grid=(grid,),
        in_specs=[
            pl.BlockSpec((Bn, HID), lambda i: (i, 0)),
            pl.BlockSpec((2, 2, Bn, HID), lambda i: (0, 0, i, 0)),
            full((HEADS, HID)),
            full((HID, 2 * HID)),
            full((1, 2 * HID)),
            full((2 * HID, HID)),
            full((1, HID)),
            full((1, HID)),
            full((1, HID)),
            full((1, HID)),
            full((1, HID)),
        ],
        out_specs=pl.BlockSpec((Bn, HID), lambda i: (i, 0)),
        out_shape=jax.ShapeDtypeStruct((N, HID), _f32),
    )(x, agg2, R, f1T, f1b, f2T, f2b, ln1w, ln1b, ln2w, ln2b)


def kernel(x, e, edge_index, qkv_w, qkv_b, ew_w, ew_b, aw, c1_w, c1_b,
           c2_w, c2_b, f1_w, f1_b, f2_w, f2_b, ln1h_w, ln1h_b, ln2h_w,
           ln2h_b, ln1c_w, ln1c_b, ln2c_w, ln2c_b):
    dst = edge_index[0]
    src = edge_index[1]
    r2 = lambda v: v.reshape(1, -1)
    awT = aw[:, :, 0].T
    A = (awT[:, :, None] * jnp.eye(HEADS, dtype=_f32)[:, None, :]).reshape(HID, HEADS)
    R = jnp.repeat(jnp.eye(HEADS, dtype=_f32), DH, axis=1)

    q, kv = _qkv_call(x, qkv_w.T, r2(qkv_b))
    qd, kvs = _gather_call(q, kv, dst, src)
    conn, pack, wgt = _edge_call(
        e, qd, kvs, ew_w.T, r2(ew_b), A, R, c1_w.T, r2(c1_b), c2_w.T,
        r2(c2_b), r2(ln1c_w), r2(ln1c_b), r2(ln2c_w), r2(ln2c_b))
    agg2 = _scatter_call(pack, wgt, dst).reshape(NC, 2, NP, HID)
    h = _node_call(x, agg2, R, f1_w.T, r2(f1_b), f2_w.T, r2(f2_b),
                   r2(ln1h_w), r2(ln1h_b), r2(ln2h_w), r2(ln2h_b))
    return h, conn


# speedup vs baseline: 2.4742x; 1.0187x over previous
import functools

import jax
import jax.numpy as jnp
from jax import lax
from jax.experimental import pallas as pl
from jax.experimental.pallas import tpu as pltpu
from jax.experimental.pallas import tpu_sc as plsc

N = 10000
E = 320000
HID = 128
HEADS = 8
DH = HID // HEADS
CLAMP = 5.0

NC = 2
NS = 16
NW = NC * NS
EPW = E // NW
CH = 128
NP = 10240
RPS = NP // NS
ZR = 64
NFULL = EPW // CH
TAIL = EPW - NFULL * CH
PAIRS = NFULL // 2 - 1
SCH = 64
SNFULL = EPW // SCH
STAIL = EPW - SNFULL * SCH
SPAIRS = SNFULL // 2 - 1

_f32 = jnp.float32


def _ln(x, w, b, eps=1e-5):
    mu = jnp.mean(x, axis=-1, keepdims=True)
    var = jnp.mean(x * x, axis=-1, keepdims=True) - mu * mu
    return (x - mu) * jax.lax.rsqrt(var + eps) * w + b


def _qkv_body(x_ref, w_ref, b_ref, q_ref, kv_ref):
    qkv = jnp.dot(x_ref[...], w_ref[...], preferred_element_type=_f32) + b_ref[...]
    q_ref[...] = qkv[:, :HID]
    ku = lax.bitcast_convert_type(
        qkv[:, HID:2 * HID].astype(jnp.bfloat16), jnp.uint16).astype(jnp.uint32)
    vu = lax.bitcast_convert_type(
        qkv[:, 2 * HID:].astype(jnp.bfloat16), jnp.uint16).astype(jnp.uint32)
    kv_ref[...] = lax.bitcast_convert_type((ku << 16) | vu, jnp.int32)


def _qkv_call(x, wT, b2):
    return pl.pallas_call(
        _qkv_body,
        out_shape=(
            jax.ShapeDtypeStruct((N, HID), _f32),
            jax.ShapeDtypeStruct((N, HID), jnp.int32),
        ),
    )(x, wT, b2)


def _gather_body(q_hbm, kv_hbm, dst_hbm, src_hbm, qd_hbm, kvs_hbm,
                 idx_d0, idx_s0, idx_d1, idx_s1, qb0, kvb0, qb1, kvb1,
                 gsem0, gsem1, wsem0, wsem1):
    wid = lax.axis_index("s") * NC + lax.axis_index("c")
    base = wid * EPW
    sets = ((idx_d0, idx_s0, qb0, kvb0, gsem0, wsem0),
            (idx_d1, idx_s1, qb1, kvb1, gsem1, wsem1))

    def issue_gather(c, k):
        idx_d, idx_s, qb, kvb, gsem, _ = sets[k]
        off = base + c * CH
        pltpu.sync_copy(dst_hbm.at[pl.ds(off, CH)], idx_d)
        pltpu.sync_copy(src_hbm.at[pl.ds(off, CH)], idx_s)
        pltpu.async_copy(q_hbm.at[idx_d], qb, gsem)
        pltpu.async_copy(kv_hbm.at[idx_s], kvb, gsem)

    def wait_gather(k):
        idx_d, idx_s, qb, kvb, gsem, _ = sets[k]
        pltpu.make_async_copy(q_hbm.at[idx_d], qb, gsem).wait()
        pltpu.make_async_copy(kv_hbm.at[idx_s], kvb, gsem).wait()

    def issue_wb(c, k):
        _, _, qb, kvb, _, wsem = sets[k]
        off = base + c * CH
        pltpu.async_copy(qb, qd_hbm.at[pl.ds(off, CH)], wsem)
        pltpu.async_copy(kvb, kvs_hbm.at[pl.ds(off, CH)], wsem)

    def wait_wb(c, k):
        _, _, qb, kvb, _, wsem = sets[k]
        off = base + c * CH
        pltpu.make_async_copy(qb, qd_hbm.at[pl.ds(off, CH)], wsem).wait()
        pltpu.make_async_copy(kvb, kvs_hbm.at[pl.ds(off, CH)], wsem).wait()

    issue_gather(0, 0)

    def pair(j, carry):
        a = 2 * j
        b = a + 1
        issue_gather(b, 1)
        wait_gather(0)
        issue_wb(a, 0)
        wait_gather(1)
        issue_wb(b, 1)
        wait_wb(a, 0)
        issue_gather(a + 2, 0)
        wait_wb(b, 1)
        return carry

    lax.fori_loop(0, PAIRS, pair, 0)
    issue_gather(NFULL - 1, 1)
    wait_gather(0)
    issue_wb(NFULL - 2, 0)
    wait_gather(1)
    issue_wb(NFULL - 1, 1)
    wait_wb(NFULL - 2, 0)
    toff = base + NFULL * CH
    pltpu.sync_copy(dst_hbm.at[pl.ds(toff, TAIL)], idx_d0.at[pl.ds(0, TAIL)])
    pltpu.sync_copy(src_hbm.at[pl.ds(toff, TAIL)], idx_s0.at[pl.ds(0, TAIL)])
    pltpu.async_copy(q_hbm.at[idx_d0.at[pl.ds(0, TAIL)]],
                     qb0.at[pl.ds(0, TAIL)], gsem0).wait()
    pltpu.async_copy(kv_hbm.at[idx_s0.at[pl.ds(0, TAIL)]],
                     kvb0.at[pl.ds(0, TAIL)], gsem0).wait()
    pltpu.sync_copy(qb0.at[pl.ds(0, TAIL)], qd_hbm.at[pl.ds(toff, TAIL)])
    pltpu.sync_copy(kvb0.at[pl.ds(0, TAIL)], kvs_hbm.at[pl.ds(toff, TAIL)])
    wait_wb(NFULL - 1, 1)


def _gather_call(q, kv, dst, src):
    mesh = plsc.VectorSubcoreMesh(core_axis_name="c", subcore_axis_name="s")
    fn = pl.kernel(
        _gather_body,
        out_type=(
            jax.ShapeDtypeStruct((E, HID), _f32),
            jax.ShapeDtypeStruct((E, HID), jnp.int32),
        ),
        mesh=mesh,
        scratch_types=[
            pltpu.VMEM((CH,), jnp.int32),
            pltpu.VMEM((CH,), jnp.int32),
            pltpu.VMEM((CH,), jnp.int32),
            pltpu.VMEM((CH,), jnp.int32),
            pltpu.VMEM((CH, HID), _f32),
            pltpu.VMEM((CH, HID), jnp.int32),
            pltpu.VMEM((CH, HID), _f32),
            pltpu.VMEM((CH, HID), jnp.int32),
            pltpu.SemaphoreType.DMA,
            pltpu.SemaphoreType.DMA,
            pltpu.SemaphoreType.DMA,
            pltpu.SemaphoreType.DMA,
        ],
    )
    return fn(q, kv, dst, src)


def _edge_body(e_ref, qd_ref, kvs_ref, ewT_ref, ewb_ref, A_ref, R_ref,
               c1T_ref, c1b_ref, c2T_ref, c2b_ref,
               ln1cw_ref, ln1cb_ref, ln2cw_ref, ln2cb_ref,
               conn_ref, pack_ref, wgt_ref):
    e_blk = e_ref[...]
    eh = jnp.dot(e_blk, ewT_ref[...], preferred_element_type=_f32) + ewb_ref[...]
    ew = eh[:, :HID]
    eb = eh[:, HID:]
    qd = qd_ref[...]
    kvu = lax.bitcast_convert_type(kvs_ref[...], jnp.uint32)
    ks = lax.bitcast_convert_type(
        (kvu >> 16).astype(jnp.uint16), jnp.bfloat16).astype(_f32)
    vs = lax.bitcast_convert_type(
        (kvu & 0xFFFF).astype(jnp.uint16), jnp.bfloat16).astype(_f32)
    conn1 = (qd + ks) * ew
    conn2 = jnp.sign(conn1) * jnp.sqrt(jnp.abs(conn1))
    conn = jnp.maximum(conn2 + eb, 0.0)
    score = jnp.dot(conn, A_ref[...], preferred_element_type=_f32)
    wgt = jnp.exp(jnp.clip(score, -CLAMP, CLAMP))
    connc1 = jnp.dot(conn, c1T_ref[...], preferred_element_type=_f32) + c1b_ref[...]
    msg = vs + connc1
    contrib = msg * jnp.dot(wgt, R_ref[...], preferred_element_type=_f32)
    pack_ref[...] = contrib
    wgt_ref[...] = jnp.concatenate([wgt, jnp.zeros_like(wgt)], axis=1)
    cl = jnp.maximum(_ln(connc1, ln1cw_ref[...], ln1cb_ref[...]), 0.0)
    co = jnp.dot(cl, c2T_ref[...], preferred_element_type=_f32) + c2b_ref[...] + e_blk
    conn_ref[...] = _ln(co, ln2cw_ref[...], ln2cb_ref[...])


def _edge_call(e, qd, kvs, ewT, ewb, A, R, c1T, c1b, c2T, c2b,
               ln1cw, ln1cb, ln2cw, ln2cb):
    B = 2000
    grid = E // B
    full = lambda shape: pl.BlockSpec(shape, lambda i: (0, 0))
    return pl.pallas_call(
        _edge_body,
        grid=(grid,),
        in_specs=[
            pl.BlockSpec((B, HID), lambda i: (i, 0)),
            pl.BlockSpec((B, HID), lambda i: (i, 0)),
            pl.BlockSpec((B, HID), lambda i: (i, 0)),
            full((HID, 2 * HID)),
            full((1, 2 * HID)),
            full((HID, HEADS)),
            full((HEADS, HID)),
            full((HID, HID)),
            full((1, HID)),
            full((HID, HID)),
            full((1, HID)),
            full((1, HID)),
            full((1, HID)),
            full((1, HID)),
            full((1, HID)),
        ],
        out_specs=[
            pl.BlockSpec((B, HID), lambda i: (i, 0)),
            pl.BlockSpec((B, HID), lambda i: (i, 0)),
            pl.BlockSpec((B, 2 * HEADS), lambda i: (i, 0)),
        ],
        out_shape=(
            jax.ShapeDtypeStruct((E, HID), _f32),
            jax.ShapeDtypeStruct((E, HID), _f32),
            jax.ShapeDtypeStruct((E, 2 * HEADS), _f32),
        ),
    )(e, qd, kvs, ewT, ewb, A, R, c1T, c1b, c2T, c2b,
      ln1cw, ln1cb, ln2cw, ln2cb)


def _scatter_body(pack_hbm, wgt_hbm, dst_hbm, agg_hbm,
                  idx0, idx1, idx_t, vbuf0, vbuf1, wbuf0, wbuf1,
                  zbuf, shared, lsem0, lsem1):
    c = lax.axis_index("c")
    s = lax.axis_index("s")
    wid = s * NC + c
    base = wid * EPW
    isets = (idx0, idx1)
    lsems = (lsem0, lsem1)
    vsets = (vbuf0, vbuf1)
    wsets = (wbuf0, wbuf1)

    def zrow(i, carry):
        for j in range(HID // 16):
            zbuf[i, pl.ds(j * 16, 16)] = jnp.zeros((16,), _f32)
        return carry

    lax.fori_loop(0, ZR, zrow, 0)

    def zero_shared():
        def zcp(k, carry):
            pltpu.sync_copy(zbuf, shared.at[pl.ds(s * RPS + k * ZR, ZR)])
            return carry
        lax.fori_loop(0, RPS // ZR, zcp, 0)

    def phase(val_hbm, vbufs, scatter_fn, out_row):
        def issue(ch, k):
            off = base + ch * SCH
            pltpu.async_copy(dst_hbm.at[pl.ds(off, SCH)], isets[k], lsems[k])
            pltpu.async_copy(val_hbm.at[pl.ds(off, SCH)], vbufs[k], lsems[k])

        def wait(ch, k):
            off = base + ch * SCH
            pltpu.make_async_copy(
                dst_hbm.at[pl.ds(off, SCH)], isets[k], lsems[k]).wait()
            pltpu.make_async_copy(
                val_hbm.at[pl.ds(off, SCH)], vbufs[k], lsems[k]).wait()

        zero_shared()
        plsc.subcore_barrier()
        issue(0, 0)

        def pair(j, carry):
            a = 2 * j
            issue(a + 1, 1)
            wait(a, 0)
            scatter_fn(0)
            issue(a + 2, 0)
            wait(a + 1, 1)
            scatter_fn(1)
            return carry

        lax.fori_loop(0, SPAIRS, pair, 0)
        issue(SNFULL - 1, 1)
        wait(SNFULL - 2, 0)
        scatter_fn(0)
        wait(SNFULL - 1, 1)
        scatter_fn(1)
        toff = base + SNFULL * SCH
        pltpu.sync_copy(dst_hbm.at[pl.ds(toff, STAIL)], idx_t)
        pltpu.sync_copy(val_hbm.at[pl.ds(toff, STAIL)],
                        vbufs[0].at[pl.ds(0, STAIL)])
        scatter_fn(0, STAIL)
        plsc.subcore_barrier()
        pltpu.sync_copy(shared.at[pl.ds(s * RPS, RPS)],
                        agg_hbm.at[pl.ds(out_row + s * RPS, RPS)])
        plsc.subcore_barrier()

    def scat1(k, tail=0):
        if tail:
            pltpu.sync_copy(vsets[0].at[pl.ds(0, tail)],
                            shared.at[idx_t], add=True)
        else:
            pltpu.sync_copy(vsets[k], shared.at[isets[k]], add=True)

    phase(pack_hbm, vsets, scat1, c * 2 * NP)

    def zpad(i, carry):
        for j in range(HID // 16):
            vbuf0[i, pl.ds(j * 16, 16)] = jnp.zeros((16,), _f32)
        return carry

    lax.fori_loop(0, SCH, zpad, 0)

    def scat2(k, tail=0):
        n = tail if tail else SCH
        for r in range(n):
            vbuf0[r, pl.ds(0, 16)] = wsets[k][r, pl.ds(0, 16)]
        if tail:
            pltpu.sync_copy(vbuf0.at[pl.ds(0, tail)], shared.at[idx_t], add=True)
        else:
            pltpu.sync_copy(vbuf0, shared.at[isets[k]], add=True)

    phase(wgt_hbm, wsets, scat2, (c * 2 + 1) * NP)


def _scatter_call(pack, wgt, dst):
    mesh = plsc.VectorSubcoreMesh(core_axis_name="c", subcore_axis_name="s")
    fn = pl.kernel(
        _scatter_body,
        out_type=jax.ShapeDtypeStruct((NC * 2 * NP, HID), _f32),
        mesh=mesh,
        scratch_types=[
            pltpu.VMEM((SCH,), jnp.int32),
            pltpu.VMEM((SCH,), jnp.int32),
            pltpu.VMEM((STAIL,), jnp.int32),
            pltpu.VMEM((SCH, HID), _f32),
            pltpu.VMEM((SCH, HID), _f32),
            pltpu.VMEM((SCH, 2 * HEADS), _f32),
            pltpu.VMEM((SCH, 2 * HEADS), _f32),
            pltpu.VMEM((ZR, HID), _f32),
            pltpu.VMEM_SHARED((NP, HID), _f32),
            pltpu.SemaphoreType.DMA,
            pltpu.SemaphoreType.DMA,
        ],
    )
    return fn(pack, wgt, dst)


def _node_body(x_ref, agg_ref, R_ref, f1T_ref, f1b_ref, f2T_ref, f2b_ref,
               ln1w_ref, ln1b_ref, ln2w_ref, ln2b_ref, h_ref):
    agg = agg_ref[0, 0] + agg_ref[1, 0]
    den8 = agg_ref[0, 1, :, :HEADS] + agg_ref[1, 1, :, :HEADS]
    den = jnp.dot(den8, R_ref[...], preferred_element_type=_f32)
    h = x_ref[...] + agg / (den + 1e-16)
    hres = h
    h = _ln(h, ln1w_ref[...], ln1b_ref[...])
    h = jnp.maximum(jnp.dot(h, f1T_ref[...], preferred_element_type=_f32)
                    + f1b_ref[...], 0.0)
    h = jnp.dot(h, f2T_ref[...], preferred_element_type=_f32) + f2b_ref[...] + hres
    h_ref[...] = _ln(h, ln2w_ref[...], ln2b_ref[...])


def _node_call(x, agg2, R, f1T, f1b, f2T, f2b, ln1w, ln1b, ln2w, ln2b):
    Bn = 1000
    grid = N // Bn
    full = lambda shape: pl.BlockSpec(shape, lambda i: (0,) * len(shape))
    return pl.pallas_call(
        _node_body,
        grid=(grid,),
        in_specs=[
            pl.BlockSpec((Bn, HID), lambda i: (i, 0)),
            pl.BlockSpec((2, 2, Bn, HID), lambda i: (0, 0, i, 0)),
            full((HEADS, HID)),
            full((HID, 2 * HID)),
            full((1, 2 * HID)),
            full((2 * HID, HID)),
            full((1, HID)),
            full((1, HID)),
            full((1, HID)),
            full((1, HID)),
            full((1, HID)),
        ],
        out_specs=pl.BlockSpec((Bn, HID), lambda i: (i, 0)),
        out_shape=jax.ShapeDtypeStruct((N, HID), _f32),
    )(x, agg2, R, f1T, f1b, f2T, f2b, ln1w, ln1b, ln2w, ln2b)


def kernel(x, e, edge_index, qkv_w, qkv_b, ew_w, ew_b, aw, c1_w, c1_b,
           c2_w, c2_b, f1_w, f1_b, f2_w, f2_b, ln1h_w, ln1h_b, ln2h_w,
           ln2h_b, ln1c_w, ln1c_b, ln2c_w, ln2c_b):
    dst = edge_index[0]
    src = edge_index[1]
    r2 = lambda v: v.reshape(1, -1)
    awT = aw[:, :, 0].T
    A = (awT[:, :, None] * jnp.eye(HEADS, dtype=_f32)[:, None, :]).reshape(HID, HEADS)
    R = jnp.repeat(jnp.eye(HEADS, dtype=_f32), DH, axis=1)

    q, kv = _qkv_call(x, qkv_w.T, r2(qkv_b))
    qd, kvs = _gather_call(q, kv, dst, src)
    conn, pack, wgt = _edge_call(
        e, qd, kvs, ew_w.T, r2(ew_b), A, R, c1_w.T, r2(c1_b), c2_w.T,
        r2(c2_b), r2(ln1c_w), r2(ln1c_b), r2(ln2c_w), r2(ln2c_b))
    agg2 = _scatter_call(pack, wgt, dst).reshape(NC, 2, NP, HID)
    h = _node_call(x, agg2, R, f1_w.T, r2(f1_b), f2_w.T, r2(f2_b),
                   r2(ln1h_w), r2(ln1h_b), r2(ln2h_w), r2(ln2h_b))
    return h, conn


# speedup vs baseline: 2.6667x; 1.0778x over previous
import functools

import jax
import jax.numpy as jnp
from jax import lax
from jax.experimental import pallas as pl
from jax.experimental.pallas import tpu as pltpu
from jax.experimental.pallas import tpu_sc as plsc

N = 10000
E = 320000
HID = 128
HEADS = 8
DH = HID // HEADS
CLAMP = 5.0

NC = 2
NS = 16
NW = NC * NS
CH = 128
NP = 10240
RPS = NP // NS
ZR = 64
SCH = 64
E0 = 163840
E1 = E - E0

_f32 = jnp.float32


def _ln(x, w, b, eps=1e-5):
    mu = jnp.mean(x, axis=-1, keepdims=True)
    var = jnp.mean(x * x, axis=-1, keepdims=True) - mu * mu
    return (x - mu) * jax.lax.rsqrt(var + eps) * w + b


def _qkv_body(x_ref, w_ref, b_ref, q_ref, kv_ref):
    qkv = jnp.dot(x_ref[...], w_ref[...], preferred_element_type=_f32) + b_ref[...]
    q_ref[...] = qkv[:, :HID]
    ku = lax.bitcast_convert_type(
        qkv[:, HID:2 * HID].astype(jnp.bfloat16), jnp.uint16).astype(jnp.uint32)
    vu = lax.bitcast_convert_type(
        qkv[:, 2 * HID:].astype(jnp.bfloat16), jnp.uint16).astype(jnp.uint32)
    kv_ref[...] = lax.bitcast_convert_type((ku << 16) | vu, jnp.int32)


def _qkv_call(x, wT, b2):
    return pl.pallas_call(
        _qkv_body,
        out_shape=(
            jax.ShapeDtypeStruct((N, HID), _f32),
            jax.ShapeDtypeStruct((N, HID), jnp.int32),
        ),
    )(x, wT, b2)


def _gather_body(lo, epw, q_hbm, kv_hbm, dst_hbm, src_hbm, qd_hbm, kvs_hbm,
                 idx_d0, idx_s0, idx_d1, idx_s1, qb0, kvb0, qb1, kvb1,
                 gsem0, gsem1, wsem0, wsem1):
    nfull = epw // CH
    tail = epw - nfull * CH
    pairs = nfull // 2 - 1
    wid = lax.axis_index("s") * NC + lax.axis_index("c")
    obase = wid * epw
    base = lo + obase
    sets = ((idx_d0, idx_s0, qb0, kvb0, gsem0, wsem0),
            (idx_d1, idx_s1, qb1, kvb1, gsem1, wsem1))

    def issue_gather(c, k):
        idx_d, idx_s, qb, kvb, gsem, _ = sets[k]
        off = base + c * CH
        pltpu.sync_copy(dst_hbm.at[pl.ds(off, CH)], idx_d)
        pltpu.sync_copy(src_hbm.at[pl.ds(off, CH)], idx_s)
        pltpu.async_copy(q_hbm.at[idx_d], qb, gsem)
        pltpu.async_copy(kv_hbm.at[idx_s], kvb, gsem)

    def wait_gather(k):
        idx_d, idx_s, qb, kvb, gsem, _ = sets[k]
        pltpu.make_async_copy(q_hbm.at[idx_d], qb, gsem).wait()
        pltpu.make_async_copy(kv_hbm.at[idx_s], kvb, gsem).wait()

    def issue_wb(c, k):
        _, _, qb, kvb, _, wsem = sets[k]
        off = obase + c * CH
        pltpu.async_copy(qb, qd_hbm.at[pl.ds(off, CH)], wsem)
        pltpu.async_copy(kvb, kvs_hbm.at[pl.ds(off, CH)], wsem)

    def wait_wb(c, k):
        _, _, qb, kvb, _, wsem = sets[k]
        off = obase + c * CH
        pltpu.make_async_copy(qb, qd_hbm.at[pl.ds(off, CH)], wsem).wait()
        pltpu.make_async_copy(kvb, kvs_hbm.at[pl.ds(off, CH)], wsem).wait()

    issue_gather(0, 0)

    def pair(j, carry):
        a = 2 * j
        b = a + 1
        issue_gather(b, 1)
        wait_gather(0)
        issue_wb(a, 0)
        wait_gather(1)
        issue_wb(b, 1)
        wait_wb(a, 0)
        issue_gather(a + 2, 0)
        wait_wb(b, 1)
        return carry

    lax.fori_loop(0, pairs, pair, 0)
    issue_gather(nfull - 1, 1)
    wait_gather(0)
    issue_wb(nfull - 2, 0)
    wait_gather(1)
    issue_wb(nfull - 1, 1)
    wait_wb(nfull - 2, 0)
    if tail:
        toff = base + nfull * CH
        ooff = obase + nfull * CH
        pltpu.sync_copy(dst_hbm.at[pl.ds(toff, tail)], idx_d0.at[pl.ds(0, tail)])
        pltpu.sync_copy(src_hbm.at[pl.ds(toff, tail)], idx_s0.at[pl.ds(0, tail)])
        pltpu.async_copy(q_hbm.at[idx_d0.at[pl.ds(0, tail)]],
                         qb0.at[pl.ds(0, tail)], gsem0).wait()
        pltpu.async_copy(kv_hbm.at[idx_s0.at[pl.ds(0, tail)]],
                         kvb0.at[pl.ds(0, tail)], gsem0).wait()
        pltpu.sync_copy(qb0.at[pl.ds(0, tail)], qd_hbm.at[pl.ds(ooff, tail)])
        pltpu.sync_copy(kvb0.at[pl.ds(0, tail)], kvs_hbm.at[pl.ds(ooff, tail)])
    wait_wb(nfull - 1, 1)


def _gather_call(q, kv, dst, src, lo, ne):
    mesh = plsc.VectorSubcoreMesh(core_axis_name="c", subcore_axis_name="s")
    fn = pl.kernel(
        functools.partial(_gather_body, lo, ne // NW),
        out_type=(
            jax.ShapeDtypeStruct((ne, HID), _f32),
            jax.ShapeDtypeStruct((ne, HID), jnp.int32),
        ),
        mesh=mesh,
        scratch_types=[
            pltpu.VMEM((CH,), jnp.int32),
            pltpu.VMEM((CH,), jnp.int32),
            pltpu.VMEM((CH,), jnp.int32),
            pltpu.VMEM((CH,), jnp.int32),
            pltpu.VMEM((CH, HID), _f32),
            pltpu.VMEM((CH, HID), jnp.int32),
            pltpu.VMEM((CH, HID), _f32),
            pltpu.VMEM((CH, HID), jnp.int32),
            pltpu.SemaphoreType.DMA,
            pltpu.SemaphoreType.DMA,
            pltpu.SemaphoreType.DMA,
            pltpu.SemaphoreType.DMA,
        ],
    )
    return fn(q, kv, dst, src)


def _edge_body(e_ref, qd_ref, kvs_ref, ewT_ref, ewb_ref, A_ref, R_ref,
               c1T_ref, c1b_ref, c2T_ref, c2b_ref,
               ln1cw_ref, ln1cb_ref, ln2cw_ref, ln2cb_ref,
               *rest):
    conn_ref, pack_ref, wgt_ref = rest[-3:]
    e_blk = e_ref[...]
    eh = jnp.dot(e_blk, ewT_ref[...], preferred_element_type=_f32) + ewb_ref[...]
    ew = eh[:, :HID]
    eb = eh[:, HID:]
    qd = qd_ref[...]
    kvu = lax.bitcast_convert_type(kvs_ref[...], jnp.uint32)
    ks = lax.bitcast_convert_type(
        (kvu >> 16).astype(jnp.uint16), jnp.bfloat16).astype(_f32)
    vs = lax.bitcast_convert_type(
        (kvu & 0xFFFF).astype(jnp.uint16), jnp.bfloat16).astype(_f32)
    conn1 = (qd + ks) * ew
    conn2 = jnp.sign(conn1) * jnp.sqrt(jnp.abs(conn1))
    conn = jnp.maximum(conn2 + eb, 0.0)
    score = jnp.dot(conn, A_ref[...], preferred_element_type=_f32)
    wgt = jnp.exp(jnp.clip(score, -CLAMP, CLAMP))
    connc1 = jnp.dot(conn, c1T_ref[...], preferred_element_type=_f32) + c1b_ref[...]
    msg = vs + connc1
    contrib = msg * jnp.dot(wgt, R_ref[...], preferred_element_type=_f32)
    pack_ref[...] = contrib
    wgt_ref[...] = jnp.concatenate([wgt, jnp.zeros_like(wgt)], axis=1)
    cl = jnp.maximum(_ln(connc1, ln1cw_ref[...], ln1cb_ref[...]), 0.0)
    co = jnp.dot(cl, c2T_ref[...], preferred_element_type=_f32) + c2b_ref[...] + e_blk
    conn_ref[...] = _ln(co, ln2cw_ref[...], ln2cb_ref[...])


EB = 1280


def _edge_call(e, qd, kvs, ewT, ewb, A, R, c1T, c1b, c2T, c2b,
               ln1cw, ln1cb, ln2cw, ln2cb, lo, ne, conn_in=None):
    blk0 = lo // EB
    grid = ne // EB
    full = lambda shape: pl.BlockSpec(shape, lambda i: (0, 0))
    ins = [
        pl.BlockSpec((EB, HID), lambda i: (i + blk0, 0)),
        pl.BlockSpec((EB, HID), lambda i: (i, 0)),
        pl.BlockSpec((EB, HID), lambda i: (i, 0)),
        full((HID, 2 * HID)),
        full((1, 2 * HID)),
        full((HID, HEADS)),
        full((HEADS, HID)),
        full((HID, HID)),
        full((1, HID)),
        full((HID, HID)),
        full((1, HID)),
        full((1, HID)),
        full((1, HID)),
        full((1, HID)),
        full((1, HID)),
    ]
    args = [e, qd, kvs, ewT, ewb, A, R, c1T, c1b, c2T, c2b,
            ln1cw, ln1cb, ln2cw, ln2cb]
    aliases = {}
    if conn_in is not None:
        ins.append(pl.BlockSpec(memory_space=pl.ANY))
        args.append(conn_in)
        aliases = {15: 0}
    return pl.pallas_call(
        _edge_body,
        grid=(grid,),
        in_specs=ins,
        out_specs=[
            pl.BlockSpec((EB, HID), lambda i: (i + blk0, 0)),
            pl.BlockSpec((EB, HID), lambda i: (i, 0)),
            pl.BlockSpec((EB, 2 * HEADS), lambda i: (i, 0)),
        ],
        out_shape=(
            jax.ShapeDtypeStruct((E, HID), _f32),
            jax.ShapeDtypeStruct((ne, HID), _f32),
            jax.ShapeDtypeStruct((ne, 2 * HEADS), _f32),
        ),
        input_output_aliases=aliases,
    )(*args)


def _scatter_body(lo, epw, pack_hbm, wgt_hbm, dst_hbm, agg_hbm,
                  idx0, idx1, idx_t, vbuf0, vbuf1, wbuf0, wbuf1,
                  zbuf, shared, lsem0, lsem1):
    snfull = epw // SCH
    stail = epw - snfull * SCH
    spairs = snfull // 2 - 1
    c = lax.axis_index("c")
    s = lax.axis_index("s")
    wid = s * NC + c
    vbase = wid * epw
    base = lo + vbase
    isets = (idx0, idx1)
    lsems = (lsem0, lsem1)
    vsets = (vbuf0, vbuf1)
    wsets = (wbuf0, wbuf1)

    def zrow(i, carry):
        for j in range(HID // 16):
            zbuf[i, pl.ds(j * 16, 16)] = jnp.zeros((16,), _f32)
        return carry

    lax.fori_loop(0, ZR, zrow, 0)

    def zero_shared():
        def zcp(k, carry):
            pltpu.sync_copy(zbuf, shared.at[pl.ds(s * RPS + k * ZR, ZR)])
            return carry
        lax.fori_loop(0, RPS // ZR, zcp, 0)

    def phase(val_hbm, vbufs, scatter_fn, out_row):
        def issue(ch, k):
            pltpu.async_copy(dst_hbm.at[pl.ds(base + ch * SCH, SCH)],
                             isets[k], lsems[k])
            pltpu.async_copy(val_hbm.at[pl.ds(vbase + ch * SCH, SCH)],
                             vbufs[k], lsems[k])

        def wait(ch, k):
            pltpu.make_async_copy(
                dst_hbm.at[pl.ds(base + ch * SCH, SCH)], isets[k],
                lsems[k]).wait()
            pltpu.make_async_copy(
                val_hbm.at[pl.ds(vbase + ch * SCH, SCH)], vbufs[k],
                lsems[k]).wait()

        zero_shared()
        plsc.subcore_barrier()
        issue(0, 0)

        def pair(j, carry):
            a = 2 * j
            issue(a + 1, 1)
            wait(a, 0)
            scatter_fn(0)
            issue(a + 2, 0)
            wait(a + 1, 1)
            scatter_fn(1)
            return carry

        lax.fori_loop(0, spairs, pair, 0)
        issue(snfull - 1, 1)
        wait(snfull - 2, 0)
        scatter_fn(0)
        wait(snfull - 1, 1)
        scatter_fn(1)
        if stail:
            pltpu.sync_copy(dst_hbm.at[pl.ds(base + snfull * SCH, stail)],
                            idx_t)
            pltpu.sync_copy(val_hbm.at[pl.ds(vbase + snfull * SCH, stail)],
                            vbufs[0].at[pl.ds(0, stail)])
            scatter_fn(0, stail)
        plsc.subcore_barrier()
        pltpu.sync_copy(shared.at[pl.ds(s * RPS, RPS)],
                        agg_hbm.at[pl.ds(out_row + s * RPS, RPS)])
        plsc.subcore_barrier()

    def scat1(k, tail=0):
        if tail:
            pltpu.sync_copy(vsets[0].at[pl.ds(0, tail)],
                            shared.at[idx_t], add=True)
        else:
            pltpu.sync_copy(vsets[k], shared.at[isets[k]], add=True)

    phase(pack_hbm, vsets, scat1, c * 2 * NP)

    def zpad(i, carry):
        for j in range(HID // 16):
            vbuf0[i, pl.ds(j * 16, 16)] = jnp.zeros((16,), _f32)
        return carry

    lax.fori_loop(0, SCH, zpad, 0)

    def scat2(k, tail=0):
        n = tail if tail else SCH
        for r in range(n):
            vbuf0[r, pl.ds(0, 16)] = wsets[k][r, pl.ds(0, 16)]
        if tail:
            pltpu.sync_copy(vbuf0.at[pl.ds(0, tail)], shared.at[idx_t], add=True)
        else:
            pltpu.sync_copy(vbuf0, shared.at[isets[k]], add=True)

    phase(wgt_hbm, wsets, scat2, (c * 2 + 1) * NP)


def _scatter_call(pack, wgt, dst, lo, ne):
    mesh = plsc.VectorSubcoreMesh(core_axis_name="c", subcore_axis_name="s")
    fn = pl.kernel(
        functools.partial(_scatter_body, lo, ne // NW),
        out_type=jax.ShapeDtypeStruct((NC * 2 * NP, HID), _f32),
        mesh=mesh,
        scratch_types=[
            pltpu.VMEM((SCH,), jnp.int32),
            pltpu.VMEM((SCH,), jnp.int32),
            pltpu.VMEM((16,), jnp.int32),
            pltpu.VMEM((SCH, HID), _f32),
            pltpu.VMEM((SCH, HID), _f32),
            pltpu.VMEM((SCH, 2 * HEADS), _f32),
            pltpu.VMEM((SCH, 2 * HEADS), _f32),
            pltpu.VMEM((ZR, HID), _f32),
            pltpu.VMEM_SHARED((NP, HID), _f32),
            pltpu.SemaphoreType.DMA,
            pltpu.SemaphoreType.DMA,
        ],
    )
    return fn(pack, wgt, dst)


def _node_body(x_ref, agg_ref, aggb_ref, R_ref, f1T_ref, f1b_ref, f2T_ref,
               f2b_ref, ln1w_ref, ln1b_ref, ln2w_ref, ln2b_ref, h_ref):
    agg = (agg_ref[0, 0] + agg_ref[1, 0]
           + aggb_ref[0, 0] + aggb_ref[1, 0])
    den8 = (agg_ref[0, 1, :, :HEADS] + agg_ref[1, 1, :, :HEADS]
            + aggb_ref[0, 1, :, :HEADS] + aggb_ref[1, 1, :, :HEADS])
    den = jnp.dot(den8, R_ref[...], preferred_element_type=_f32)
    h = x_ref[...] + agg / (den + 1e-16)
    hres = h
    h = _ln(h, ln1w_ref[...], ln1b_ref[...])
    h = jnp.maximum(jnp.dot(h, f1T_ref[...], preferred_element_type=_f32)
                    + f1b_ref[...], 0.0)
    h = jnp.dot(h, f2T_ref[...], preferred_element_type=_f32) + f2b_ref[...] + hres
    h_ref[...] = _ln(h, ln2w_ref[...], ln2b_ref[...])


def _node_call(x, agg2, agg2b, R, f1T, f1b, f2T, f2b, ln1w, ln1b, ln2w, ln2b):
    Bn = 1000
    grid = N // Bn
    full = lambda shape: pl.BlockSpec(shape, lambda i: (0,) * len(shape))
    return pl.pallas_call(
        _node_body,
        grid=(grid,),
        in_specs=[
            pl.BlockSpec((Bn, HID), lambda i: (i, 0)),
            pl.BlockSpec((2, 2, Bn, HID), lambda i: (0, 0, i, 0)),
            pl.BlockSpec((2, 2, Bn, HID), lambda i: (0, 0, i, 0)),
            full((HEADS, HID)),
            full((HID, 2 * HID)),
            full((1, 2 * HID)),
            full((2 * HID, HID)),
            full((1, HID)),
            full((1, HID)),
            full((1, HID)),
            full((1, HID)),
            full((1, HID)),
        ],
        out_specs=pl.BlockSpec((Bn, HID), lambda i: (i, 0)),
        out_shape=jax.ShapeDtypeStruct((N, HID), _f32),
    )(x, agg2, agg2b, R, f1T, f1b, f2T, f2b, ln1w, ln1b, ln2w, ln2b)


def kernel(x, e, edge_index, qkv_w, qkv_b, ew_w, ew_b, aw, c1_w, c1_b,
           c2_w, c2_b, f1_w, f1_b, f2_w, f2_b, ln1h_w, ln1h_b, ln2h_w,
           ln2h_b, ln1c_w, ln1c_b, ln2c_w, ln2c_b):
    dst = edge_index[0]
    src = edge_index[1]
    r2 = lambda v: v.reshape(1, -1)
    awT = aw[:, :, 0].T
    A = (awT[:, :, None] * jnp.eye(HEADS, dtype=_f32)[:, None, :]).reshape(HID, HEADS)
    R = jnp.repeat(jnp.eye(HEADS, dtype=_f32), DH, axis=1)

    q, kv = _qkv_call(x, qkv_w.T, r2(qkv_b))
    ewts = (ew_w.T, r2(ew_b), A, R, c1_w.T, r2(c1_b), c2_w.T,
            r2(c2_b), r2(ln1c_w), r2(ln1c_b), r2(ln2c_w), r2(ln2c_b))
    qd0, kvs0 = _gather_call(q, kv, dst, src, 0, E0)
    qd1, kvs1 = _gather_call(q, kv, dst, src, E0, E1)
    conn0, pack0, wgt0 = _edge_call(e, qd0, kvs0, *ewts, lo=0, ne=E0)
    agg2a = _scatter_call(pack0, wgt0, dst, 0, E0).reshape(NC, 2, NP, HID)
    conn, pack1, wgt1 = _edge_call(e, qd1, kvs1, *ewts, lo=E0, ne=E1,
                                   conn_in=conn0)
    agg2b = _scatter_call(pack1, wgt1, dst, E0, E1).reshape(NC, 2, NP, HID)
    h = _node_call(x, agg2a, agg2b, R, f1_w.T, r2(f1_b), f2_w.T, r2(f2_b),
                   r2(ln1h_w), r2(ln1h_b), r2(ln2h_w), r2(ln2h_b))
    return h, conn
